# baseline (speedup 1.0000x reference)
import functools

import jax
import jax.numpy as jnp
from jax import lax
from jax.experimental import pallas as pl
from jax.experimental.pallas import tpu as pltpu
from jax.experimental.pallas import tpu_sc as plsc

HIDDEN = 64
LANES = 16
IDX_MINOR = 128
SUBGATHERS = 4
CHUNK = IDX_MINOR * SUBGATHERS


def _gather_add(table, idx2d, pos_row, n_rows):
    info = plsc.get_sparse_core_info()
    nc, ns = info.num_cores, info.num_subcores
    nw = nc * ns
    per_w = n_rows // nw
    assert per_w * nw == n_rows and per_w % CHUNK == 0
    n_chunks = per_w // CHUNK
    assert n_chunks % 2 == 0 and n_chunks >= 4
    idx_rows_per_chunk = CHUNK // IDX_MINOR

    mesh = plsc.VectorSubcoreMesh(core_axis_name="c", subcore_axis_name="s")

    @functools.partial(
        pl.kernel,
        out_type=jax.ShapeDtypeStruct((n_rows, HIDDEN), jnp.float32),
        mesh=mesh,
        compiler_params=pltpu.CompilerParams(use_tc_tiling_on_sc=False),
        scratch_types=[
            pltpu.VMEM((idx_rows_per_chunk, IDX_MINOR), jnp.int32),
            pltpu.VMEM((idx_rows_per_chunk, IDX_MINOR), jnp.int32),
            pltpu.VMEM((CHUNK, HIDDEN), jnp.float32),
            pltpu.VMEM((CHUNK, HIDDEN), jnp.float32),
            pltpu.VMEM((HIDDEN,), jnp.float32),
            pltpu.SemaphoreType.DMA,
            pltpu.SemaphoreType.DMA,
            pltpu.SemaphoreType.DMA,
            pltpu.SemaphoreType.DMA,
        ],
    )
    def k(table_hbm, idx_hbm, pos_hbm, out_hbm,
          idx_v0, idx_v1, rows_v0, rows_v1, pos_v,
          gsem0, gsem1, osem0, osem1):
        wid = lax.axis_index("s") * nc + lax.axis_index("c")
        base = wid * per_w
        ibase = wid * (per_w // IDX_MINOR)

        idx_bufs = (idx_v0, idx_v1)
        row_bufs = (rows_v0, rows_v1)
        gsems = (gsem0, gsem1)
        osems = (osem0, osem1)

        pltpu.sync_copy(pos_hbm, pos_v)
        pvecs = [pos_v[pl.ds(j * LANES, LANES)] for j in range(HIDDEN // LANES)]

        def load_idx(c, b):
            pltpu.sync_copy(
                idx_hbm.at[pl.ds(ibase + c * idx_rows_per_chunk,
                                 idx_rows_per_chunk)],
                idx_bufs[b])

        def fire_gathers(b):
            for j in range(SUBGATHERS):
                pltpu.async_copy(
                    table_hbm.at[idx_bufs[b].at[j]],
                    row_bufs[b].at[pl.ds(j * IDX_MINOR, IDX_MINOR)],
                    gsems[b])

        def wait_gathers(b):
            for j in range(SUBGATHERS):
                pltpu.make_async_copy(
                    table_hbm.at[idx_bufs[b].at[j]],
                    row_bufs[b].at[pl.ds(j * IDX_MINOR, IDX_MINOR)],
                    gsems[b]).wait()

        def add_pos(b):
            rows = row_bufs[b]

            @plsc.parallel_loop(0, CHUNK, unroll=8)
            def _(r):
                for j in range(HIDDEN // LANES):
                    sl = pl.ds(j * LANES, LANES)
                    rows[r, sl] = rows[r, sl] + pvecs[j]

        def fire_out(c, b):
            pltpu.async_copy(row_bufs[b],
                             out_hbm.at[pl.ds(base + c * CHUNK, CHUNK)],
                             osems[b])

        def wait_out(c, b):
            pltpu.make_async_copy(row_bufs[b],
                                  out_hbm.at[pl.ds(base + c * CHUNK, CHUNK)],
                                  osems[b]).wait()

        for b in range(2):
            load_idx(b, b)
            fire_gathers(b)

        def step(c, b, prefetch):
            wait_gathers(b)
            add_pos(b)
            fire_out(c, b)
            if prefetch:
                wait_out(c, b)
                load_idx(c + 2, b)
                fire_gathers(b)

        def body(i, carry):
            c0 = i * 2
            for b in range(2):
                step(c0 + b, b, prefetch=True)
            return carry

        lax.fori_loop(0, (n_chunks - 2) // 2, body, 0)

        for b in range(2):
            c = n_chunks - 2 + b
            step(c, b, prefetch=False)
            wait_out(c, b)

    return k(table, idx2d, pos_row)


def kernel(input_id, embedding, pos_code):
    batch, seq = input_id.shape
    n_rows = batch * seq
    idx2d = input_id.reshape(n_rows // IDX_MINOR, IDX_MINOR)
    pos_row = pos_code[0, seq, :]
    out = _gather_add(embedding, idx2d, pos_row, n_rows)
    return out.reshape(batch, seq, HIDDEN)


# speedup vs baseline: 1.0099x; 1.0099x over previous
import functools

import jax
import jax.numpy as jnp
from jax import lax
from jax.experimental import pallas as pl
from jax.experimental.pallas import tpu as pltpu
from jax.experimental.pallas import tpu_sc as plsc

HIDDEN = 64
LANES = 16
NB = 4


def kernel(input_id, embedding, pos_code):
    batch, seq = input_id.shape
    info = plsc.get_sparse_core_info()
    nc, ns = info.num_cores, info.num_subcores
    nw = nc * ns
    rows_per_w = batch // nw
    assert rows_per_w * nw == batch and rows_per_w % NB == 0
    n_chunks = rows_per_w // NB
    assert n_chunks % 2 == 0 and n_chunks >= 4
    splits = []
    off = 0
    while off < seq:
        w = min(128, seq - off)
        splits.append((off, w))
        off += w
    for off, _ in splits:
        assert off % 8 == 0

    mesh = plsc.VectorSubcoreMesh(core_axis_name="c", subcore_axis_name="s")

    @functools.partial(
        pl.kernel,
        out_type=jax.ShapeDtypeStruct((batch, seq, HIDDEN), jnp.float32),
        mesh=mesh,
        compiler_params=pltpu.CompilerParams(use_tc_tiling_on_sc=False),
        scratch_types=[
            pltpu.VMEM((NB, seq), jnp.int32),
            pltpu.VMEM((NB, seq), jnp.int32),
            pltpu.VMEM((NB, seq, HIDDEN), jnp.float32),
            pltpu.VMEM((NB, seq, HIDDEN), jnp.float32),
            pltpu.VMEM((HIDDEN,), jnp.float32),
            pltpu.SemaphoreType.DMA,
            pltpu.SemaphoreType.DMA,
            pltpu.SemaphoreType.DMA,
            pltpu.SemaphoreType.DMA,
        ],
    )
    def k(idx_hbm, table_hbm, pc_hbm, out_hbm,
          idx_v0, idx_v1, rows_v0, rows_v1, pos_v,
          gsem0, gsem1, osem0, osem1):
        wid = lax.axis_index("s") * nc + lax.axis_index("c")
        base = wid * rows_per_w

        idx_bufs = (idx_v0, idx_v1)
        row_bufs = (rows_v0, rows_v1)
        gsems = (gsem0, gsem1)
        osems = (osem0, osem1)

        pltpu.sync_copy(pc_hbm.at[0, seq], pos_v)
        pvecs = [pos_v[pl.ds(j * LANES, LANES)] for j in range(HIDDEN // LANES)]

        def load_idx(c, b):
            pltpu.sync_copy(idx_hbm.at[pl.ds(base + c * NB, NB)], idx_bufs[b])

        def each_gather(b):
            for j in range(NB):
                for off, w in splits:
                    yield (table_hbm.at[idx_bufs[b].at[j, pl.ds(off, w)]],
                           row_bufs[b].at[j, pl.ds(off, w)],
                           gsems[b])

        def fire_gathers(b):
            for src, dst, sem in each_gather(b):
                pltpu.async_copy(src, dst, sem)

        def wait_gathers(b):
            for src, dst, sem in each_gather(b):
                pltpu.make_async_copy(src, dst, sem).wait()

        def add_pos(b):
            rows = row_bufs[b]
            for j in range(NB):
                @plsc.parallel_loop(0, seq, unroll=8)
                def _(r):
                    for i in range(HIDDEN // LANES):
                        sl = pl.ds(i * LANES, LANES)
                        rows[j, r, sl] = rows[j, r, sl] + pvecs[i]

        def fire_out(c, b):
            pltpu.async_copy(row_bufs[b],
                             out_hbm.at[pl.ds(base + c * NB, NB)],
                             osems[b])

        def wait_out(c, b):
            pltpu.make_async_copy(row_bufs[b],
                                  out_hbm.at[pl.ds(base + c * NB, NB)],
                                  osems[b]).wait()

        for b in range(2):
            load_idx(b, b)
            fire_gathers(b)

        def step(c, b, prefetch):
            wait_gathers(b)
            add_pos(b)
            fire_out(c, b)
            if prefetch:
                wait_out(c, b)
                load_idx(c + 2, b)
                fire_gathers(b)

        def body(i, carry):
            c0 = i * 2
            for b in range(2):
                step(c0 + b, b, prefetch=True)
            return carry

        lax.fori_loop(0, (n_chunks - 2) // 2, body, 0)

        for b in range(2):
            c = n_chunks - 2 + b
            step(c, b, prefetch=False)
            wait_out(c, b)

    return k(input_id, embedding, pos_code)
